# Initial kernel scaffold; baseline (speedup 1.0000x reference)
import jax
import jax.numpy as jnp
from jax.experimental import pallas as pl


def kernel(x, edge_index, W1, b1, W2, b2):
    raise NotImplementedError("write your pallas kernel here")


# speedup vs baseline: 29.5096x; 29.5096x over previous
import functools

import jax
import jax.numpy as jnp
from jax import lax
from jax.experimental import pallas as pl
from jax.experimental.pallas import tpu as pltpu
from jax.experimental.pallas import tpu_sc as plsc

_N = 10000
_E = 320000
_DIN = 128
_DH = 16
_DOUT = 7
_NC = 2
_NS = 16
_NW = _NC * _NS
_CHUNK = 128
_CHUNKS_PER_W = 80
_EPAD = _NW * _CHUNKS_PER_W * _CHUNK
_ACC_ROWS = 10112
_ZROWS_PER_SUB = _ACC_ROWS // _NS
_OROWS_PER_SUB = _ACC_ROWS // _NS

_MESH = plsc.VectorSubcoreMesh(core_axis_name="c", subcore_axis_name="s")
_SC_PARAMS = pltpu.CompilerParams(use_tc_tiling_on_sc=False)


def _zero_acc_slice(zbuf, acc_sh, sid):
    def zbody(i, carry):
        zbuf[i, :] = jnp.zeros((_DH,), jnp.float32)
        return carry
    lax.fori_loop(0, _ZROWS_PER_SUB, zbody, None)
    pltpu.sync_copy(zbuf, acc_sh.at[pl.ds(sid * _ZROWS_PER_SUB, _ZROWS_PER_SUB)])


def _write_out_slice(acc_sh, out_hbm, cid, sid):
    pltpu.sync_copy(
        acc_sh.at[pl.ds(sid * _OROWS_PER_SUB, _OROWS_PER_SUB)],
        out_hbm.at[cid, pl.ds(sid * _OROWS_PER_SUB, _OROWS_PER_SUB)],
    )


def _sc_deg_body(dst_hbm, out_hbm, idx_d, ones_v, zbuf, acc_sh):
    cid = lax.axis_index("c")
    sid = lax.axis_index("s")
    wid = cid * _NS + sid
    _zero_acc_slice(zbuf, acc_sh, sid)

    def obody(i, carry):
        ones_v[i, :] = jnp.ones((_DH,), jnp.float32)
        return carry
    lax.fori_loop(0, _CHUNK, obody, None)
    pltpu.sync_copy(dst_hbm.at[wid], idx_d)
    plsc.subcore_barrier()

    def cbody(j, carry):
        pltpu.sync_copy(ones_v, acc_sh.at[idx_d.at[j]], add=True)
        return carry
    lax.fori_loop(0, _CHUNKS_PER_W, cbody, None)
    plsc.subcore_barrier()
    _write_out_slice(acc_sh, out_hbm, cid, sid)


_sc_deg = functools.partial(
    pl.kernel,
    out_type=jax.ShapeDtypeStruct((_NC, _ACC_ROWS, _DH), jnp.float32),
    mesh=_MESH,
    compiler_params=_SC_PARAMS,
    scratch_types=[
        pltpu.VMEM((_CHUNKS_PER_W, _CHUNK), jnp.int32),
        pltpu.VMEM((_CHUNK, _DH), jnp.float32),
        pltpu.VMEM((_ZROWS_PER_SUB, _DH), jnp.float32),
        pltpu.VMEM_SHARED((_ACC_ROWS, _DH), jnp.float32),
    ],
)(_sc_deg_body)


def _sc_prop_body(src_hbm, dst_hbm, h_hbm, out_hbm, idx_s, idx_d, rows, zbuf,
                  acc_sh, sem):
    cid = lax.axis_index("c")
    sid = lax.axis_index("s")
    wid = cid * _NS + sid
    _zero_acc_slice(zbuf, acc_sh, sid)
    pltpu.sync_copy(src_hbm.at[wid], idx_s)
    pltpu.sync_copy(dst_hbm.at[wid], idx_d)
    plsc.subcore_barrier()

    def cbody(j, carry):
        pltpu.async_copy(h_hbm.at[idx_s.at[j]], rows, sem).wait()
        pltpu.sync_copy(rows, acc_sh.at[idx_d.at[j]], add=True)
        return carry
    lax.fori_loop(0, _CHUNKS_PER_W, cbody, None)
    plsc.subcore_barrier()
    _write_out_slice(acc_sh, out_hbm, cid, sid)


_sc_prop = functools.partial(
    pl.kernel,
    out_type=jax.ShapeDtypeStruct((_NC, _ACC_ROWS, _DH), jnp.float32),
    mesh=_MESH,
    compiler_params=_SC_PARAMS,
    scratch_types=[
        pltpu.VMEM((_CHUNKS_PER_W, _CHUNK), jnp.int32),
        pltpu.VMEM((_CHUNKS_PER_W, _CHUNK), jnp.int32),
        pltpu.VMEM((_CHUNK, _DH), jnp.float32),
        pltpu.VMEM((_ZROWS_PER_SUB, _DH), jnp.float32),
        pltpu.VMEM_SHARED((_ACC_ROWS, _DH), jnp.float32),
        pltpu.SemaphoreType.DMA,
    ],
)(_sc_prop_body)


def _tc_mm1_body(x_ref, w_ref, o_ref):
    o_ref[...] = jnp.dot(x_ref[...], w_ref[...],
                         preferred_element_type=jnp.float32)


_tc_mm1 = pl.pallas_call(
    _tc_mm1_body,
    out_shape=jax.ShapeDtypeStruct((_N, _DH), jnp.float32),
)


def _tc_scale1_body(h_ref, degp_ref, ht_ref, dis_ref):
    deg = degp_ref[0, :_N, 0:1] + degp_ref[1, :_N, 0:1] + 1.0
    dis = lax.rsqrt(deg)
    dis_ref[...] = dis
    ht_ref[...] = h_ref[...] * dis


_tc_scale1 = pl.pallas_call(
    _tc_scale1_body,
    out_shape=(
        jax.ShapeDtypeStruct((_N, _DH), jnp.float32),
        jax.ShapeDtypeStruct((_N, 1), jnp.float32),
    ),
)


def _tc_mid_body(accp_ref, ht1_ref, dis_ref, b1_ref, ht2_ref):
    dis = dis_ref[...]
    z = dis * (accp_ref[0, :_N] + accp_ref[1, :_N] + ht1_ref[...]) + b1_ref[...]
    ht2_ref[...] = dis * jnp.maximum(z, 0.0)


_tc_mid = pl.pallas_call(
    _tc_mid_body,
    out_shape=jax.ShapeDtypeStruct((_N, _DH), jnp.float32),
)


def _tc_fin_body(accp_ref, ht2_ref, dis_ref, w2_ref, b2_ref, o_ref):
    t = dis_ref[...] * (accp_ref[0, :_N] + accp_ref[1, :_N] + ht2_ref[...])
    logits = jnp.dot(t, w2_ref[...], preferred_element_type=jnp.float32)
    logits = logits + b2_ref[...]
    m = jnp.max(logits, axis=1, keepdims=True)
    e = jnp.exp(logits - m)
    s = jnp.sum(e, axis=1, keepdims=True)
    o_ref[...] = logits - m - jnp.log(s)


_tc_fin = pl.pallas_call(
    _tc_fin_body,
    out_shape=jax.ShapeDtypeStruct((_N, _DOUT), jnp.float32),
)


def kernel(x, edge_index, W1, b1, W2, b2):
    ei = edge_index.astype(jnp.int32)
    pad = _EPAD - _E
    src = jnp.concatenate([ei[0], jnp.zeros((pad,), jnp.int32)])
    dst = jnp.concatenate([ei[1], jnp.full((pad,), _N, jnp.int32)])
    src3 = src.reshape(_NW, _CHUNKS_PER_W, _CHUNK)
    dst3 = dst.reshape(_NW, _CHUNKS_PER_W, _CHUNK)

    deg_parts = _sc_deg(dst3)
    h1 = _tc_mm1(x, W1)
    ht1, dis = _tc_scale1(h1, deg_parts)
    acc1 = _sc_prop(src3, dst3, ht1)
    ht2 = _tc_mid(acc1, ht1, dis, b1.reshape(1, _DH))
    acc2 = _sc_prop(src3, dst3, ht2)
    return _tc_fin(acc2, ht2, dis, W2, b2.reshape(1, _DOUT))


# speedup vs baseline: 35.3219x; 1.1970x over previous
import functools

import jax
import jax.numpy as jnp
from jax import lax
from jax.experimental import pallas as pl
from jax.experimental.pallas import tpu as pltpu
from jax.experimental.pallas import tpu_sc as plsc

_N = 10000
_E = 320000
_DIN = 128
_DH = 16
_DOUT = 7
_NC = 2
_NS = 16
_NW = _NC * _NS
_CHUNK = 2048
_CHUNKS_PER_W = 5
_EPAD = _NW * _CHUNKS_PER_W * _CHUNK
_ACC_ROWS = 10112
_ZROWS_PER_SUB = _ACC_ROWS // _NS
_OROWS_PER_SUB = _ACC_ROWS // _NS

_MESH = plsc.VectorSubcoreMesh(core_axis_name="c", subcore_axis_name="s")
_SC_PARAMS = pltpu.CompilerParams(use_tc_tiling_on_sc=False)


def _zero_acc_slice(zbuf, acc_sh, sid):
    def zbody(i, carry):
        zbuf[i, :] = jnp.zeros((_DH,), jnp.float32)
        return carry
    lax.fori_loop(0, _ZROWS_PER_SUB, zbody, None)
    pltpu.sync_copy(zbuf, acc_sh.at[pl.ds(sid * _ZROWS_PER_SUB, _ZROWS_PER_SUB)])


def _write_out_slice(acc_sh, out_hbm, cid, sid):
    pltpu.sync_copy(
        acc_sh.at[pl.ds(sid * _OROWS_PER_SUB, _OROWS_PER_SUB)],
        out_hbm.at[cid, pl.ds(sid * _OROWS_PER_SUB, _OROWS_PER_SUB)],
    )


def _sc_deg_body(dst_hbm, out_hbm, idx_d, ones_v, zbuf, acc_sh):
    cid = lax.axis_index("c")
    sid = lax.axis_index("s")
    wid = cid * _NS + sid
    _zero_acc_slice(zbuf, acc_sh, sid)

    def obody(i, carry):
        ones_v[i, :] = jnp.ones((_DH,), jnp.float32)
        return carry
    lax.fori_loop(0, _CHUNK, obody, None)
    pltpu.sync_copy(dst_hbm.at[wid], idx_d)
    plsc.subcore_barrier()

    def cbody(j, carry):
        pltpu.sync_copy(ones_v, acc_sh.at[idx_d.at[j]], add=True)
        return carry
    lax.fori_loop(0, _CHUNKS_PER_W, cbody, None)
    plsc.subcore_barrier()
    _write_out_slice(acc_sh, out_hbm, cid, sid)


_sc_deg = functools.partial(
    pl.kernel,
    out_type=jax.ShapeDtypeStruct((_NC, _ACC_ROWS, _DH), jnp.float32),
    mesh=_MESH,
    compiler_params=_SC_PARAMS,
    scratch_types=[
        pltpu.VMEM((_CHUNKS_PER_W, _CHUNK), jnp.int32),
        pltpu.VMEM((_CHUNK, _DH), jnp.float32),
        pltpu.VMEM((_ZROWS_PER_SUB, _DH), jnp.float32),
        pltpu.VMEM_SHARED((_ACC_ROWS, _DH), jnp.float32),
    ],
)(_sc_deg_body)


def _sc_prop_body(src_hbm, dst_hbm, h_hbm, out_hbm, idx_s, idx_d, rows, zbuf,
                  acc_sh, sem):
    cid = lax.axis_index("c")
    sid = lax.axis_index("s")
    wid = cid * _NS + sid
    _zero_acc_slice(zbuf, acc_sh, sid)
    pltpu.sync_copy(src_hbm.at[wid], idx_s)
    pltpu.sync_copy(dst_hbm.at[wid], idx_d)
    plsc.subcore_barrier()

    def cbody(j, carry):
        pltpu.async_copy(h_hbm.at[idx_s.at[j]], rows, sem).wait()
        pltpu.sync_copy(rows, acc_sh.at[idx_d.at[j]], add=True)
        return carry
    lax.fori_loop(0, _CHUNKS_PER_W, cbody, None)
    plsc.subcore_barrier()
    _write_out_slice(acc_sh, out_hbm, cid, sid)


_sc_prop = functools.partial(
    pl.kernel,
    out_type=jax.ShapeDtypeStruct((_NC, _ACC_ROWS, _DH), jnp.float32),
    mesh=_MESH,
    compiler_params=_SC_PARAMS,
    scratch_types=[
        pltpu.VMEM((_CHUNKS_PER_W, _CHUNK), jnp.int32),
        pltpu.VMEM((_CHUNKS_PER_W, _CHUNK), jnp.int32),
        pltpu.VMEM((_CHUNK, _DH), jnp.float32),
        pltpu.VMEM((_ZROWS_PER_SUB, _DH), jnp.float32),
        pltpu.VMEM_SHARED((_ACC_ROWS, _DH), jnp.float32),
        pltpu.SemaphoreType.DMA,
    ],
)(_sc_prop_body)


def _tc_mm1_body(x_ref, w_ref, o_ref):
    o_ref[...] = jnp.dot(x_ref[...], w_ref[...],
                         preferred_element_type=jnp.float32)


_tc_mm1 = pl.pallas_call(
    _tc_mm1_body,
    out_shape=jax.ShapeDtypeStruct((_N, _DH), jnp.float32),
)


def _tc_scale1_body(h_ref, degp_ref, ht_ref, dis_ref):
    deg = degp_ref[0, :_N, 0:1] + degp_ref[1, :_N, 0:1] + 1.0
    dis = lax.rsqrt(deg)
    dis_ref[...] = dis
    ht_ref[...] = h_ref[...] * dis


_tc_scale1 = pl.pallas_call(
    _tc_scale1_body,
    out_shape=(
        jax.ShapeDtypeStruct((_N, _DH), jnp.float32),
        jax.ShapeDtypeStruct((_N, 1), jnp.float32),
    ),
)


def _tc_mid_body(accp_ref, ht1_ref, dis_ref, b1_ref, ht2_ref):
    dis = dis_ref[...]
    z = dis * (accp_ref[0, :_N] + accp_ref[1, :_N] + ht1_ref[...]) + b1_ref[...]
    ht2_ref[...] = dis * jnp.maximum(z, 0.0)


_tc_mid = pl.pallas_call(
    _tc_mid_body,
    out_shape=jax.ShapeDtypeStruct((_N, _DH), jnp.float32),
)


def _tc_fin_body(accp_ref, ht2_ref, dis_ref, w2_ref, b2_ref, o_ref):
    t = dis_ref[...] * (accp_ref[0, :_N] + accp_ref[1, :_N] + ht2_ref[...])
    logits = jnp.dot(t, w2_ref[...], preferred_element_type=jnp.float32)
    logits = logits + b2_ref[...]
    m = jnp.max(logits, axis=1, keepdims=True)
    e = jnp.exp(logits - m)
    s = jnp.sum(e, axis=1, keepdims=True)
    o_ref[...] = logits - m - jnp.log(s)


_tc_fin = pl.pallas_call(
    _tc_fin_body,
    out_shape=jax.ShapeDtypeStruct((_N, _DOUT), jnp.float32),
)


def kernel(x, edge_index, W1, b1, W2, b2):
    ei = edge_index.astype(jnp.int32)
    pad = _EPAD - _E
    src = jnp.concatenate([ei[0], jnp.zeros((pad,), jnp.int32)])
    dst = jnp.concatenate([ei[1], jnp.full((pad,), _N, jnp.int32)])
    src3 = src.reshape(_NW, _CHUNKS_PER_W, _CHUNK)
    dst3 = dst.reshape(_NW, _CHUNKS_PER_W, _CHUNK)

    deg_parts = _sc_deg(dst3)
    h1 = _tc_mm1(x, W1)
    ht1, dis = _tc_scale1(h1, deg_parts)
    acc1 = _sc_prop(src3, dst3, ht1)
    ht2 = _tc_mid(acc1, ht1, dis, b1.reshape(1, _DH))
    acc2 = _sc_prop(src3, dst3, ht2)
    return _tc_fin(acc2, ht2, dis, W2, b2.reshape(1, _DOUT))


# speedup vs baseline: 36.7286x; 1.0398x over previous
import functools

import jax
import jax.numpy as jnp
from jax import lax
from jax.experimental import pallas as pl
from jax.experimental.pallas import tpu as pltpu
from jax.experimental.pallas import tpu_sc as plsc

_N = 10000
_E = 320000
_DIN = 128
_DH = 16
_DOUT = 7
_NC = 2
_NS = 16
_NW = _NC * _NS
_CHUNK = 2560
_CHUNKS_PER_W = 4
_EPAD = _NW * _CHUNKS_PER_W * _CHUNK
_ACC_ROWS = 10112
_ZROWS_PER_SUB = _ACC_ROWS // _NS
_OROWS_PER_SUB = _ACC_ROWS // _NS

_MESH = plsc.VectorSubcoreMesh(core_axis_name="c", subcore_axis_name="s")
_SC_PARAMS = pltpu.CompilerParams(use_tc_tiling_on_sc=False)


def _zero_acc_slice(zbuf, acc_sh, sid):
    def zbody(i, carry):
        zbuf[i, :] = jnp.zeros((_DH,), jnp.float32)
        return carry
    lax.fori_loop(0, _ZROWS_PER_SUB, zbody, None)
    pltpu.sync_copy(zbuf, acc_sh.at[pl.ds(sid * _ZROWS_PER_SUB, _ZROWS_PER_SUB)])


def _write_out_slice(acc_sh, out_hbm, cid, sid):
    pltpu.sync_copy(
        acc_sh.at[pl.ds(sid * _OROWS_PER_SUB, _OROWS_PER_SUB)],
        out_hbm.at[cid, pl.ds(sid * _OROWS_PER_SUB, _OROWS_PER_SUB)],
    )


def _sc_deg_body(dst_hbm, out_hbm, idx_d, ones_v, zbuf, acc_sh, dsem):
    cid = lax.axis_index("c")
    sid = lax.axis_index("s")
    wid = cid * _NS + sid
    _zero_acc_slice(zbuf, acc_sh, sid)

    def obody(i, carry):
        ones_v[i, :] = jnp.ones((_DH,), jnp.float32)
        return carry
    lax.fori_loop(0, _CHUNK, obody, None)
    pltpu.sync_copy(dst_hbm.at[wid], idx_d)
    plsc.subcore_barrier()

    for j in range(_CHUNKS_PER_W):
        pltpu.async_copy(ones_v, acc_sh.at[idx_d.at[j]], dsem, add=True)
    for j in range(_CHUNKS_PER_W):
        pltpu.make_async_copy(ones_v, acc_sh.at[idx_d.at[j]], dsem).wait()
    plsc.subcore_barrier()
    _write_out_slice(acc_sh, out_hbm, cid, sid)


_sc_deg = functools.partial(
    pl.kernel,
    out_type=jax.ShapeDtypeStruct((_NC, _ACC_ROWS, _DH), jnp.float32),
    mesh=_MESH,
    compiler_params=_SC_PARAMS,
    scratch_types=[
        pltpu.VMEM((_CHUNKS_PER_W, _CHUNK), jnp.int32),
        pltpu.VMEM((_CHUNK, _DH), jnp.float32),
        pltpu.VMEM((_ZROWS_PER_SUB, _DH), jnp.float32),
        pltpu.VMEM_SHARED((_ACC_ROWS, _DH), jnp.float32),
        pltpu.SemaphoreType.DMA,
    ],
)(_sc_deg_body)


def _sc_prop_body(src_hbm, dst_hbm, h_hbm, out_hbm, idx_s, idx_d, rows_a,
                  rows_b, zbuf, acc_sh, gsem_a, gsem_b, ssem_a, ssem_b):
    cid = lax.axis_index("c")
    sid = lax.axis_index("s")
    wid = cid * _NS + sid
    _zero_acc_slice(zbuf, acc_sh, sid)
    pltpu.sync_copy(src_hbm.at[wid], idx_s)
    pltpu.sync_copy(dst_hbm.at[wid], idx_d)
    plsc.subcore_barrier()

    bufs = (rows_a, rows_b)
    gsems = (gsem_a, gsem_b)
    ssems = (ssem_a, ssem_b)
    nch = _CHUNKS_PER_W
    pltpu.async_copy(h_hbm.at[idx_s.at[0]], bufs[0], gsems[0])
    for j in range(nch):
        b = j % 2
        pltpu.make_async_copy(h_hbm.at[idx_s.at[j]], bufs[b], gsems[b]).wait()
        pltpu.async_copy(bufs[b], acc_sh.at[idx_d.at[j]], ssems[b], add=True)
        if j + 1 < nch:
            nb = (j + 1) % 2
            if j - 1 >= 0:
                pltpu.make_async_copy(
                    bufs[nb], acc_sh.at[idx_d.at[j - 1]], ssems[nb]).wait()
            pltpu.async_copy(h_hbm.at[idx_s.at[j + 1]], bufs[nb], gsems[nb])
    for j in (nch - 2, nch - 1):
        pltpu.make_async_copy(
            bufs[j % 2], acc_sh.at[idx_d.at[j]], ssems[j % 2]).wait()
    plsc.subcore_barrier()
    _write_out_slice(acc_sh, out_hbm, cid, sid)


_sc_prop = functools.partial(
    pl.kernel,
    out_type=jax.ShapeDtypeStruct((_NC, _ACC_ROWS, _DH), jnp.float32),
    mesh=_MESH,
    compiler_params=_SC_PARAMS,
    scratch_types=[
        pltpu.VMEM((_CHUNKS_PER_W, _CHUNK), jnp.int32),
        pltpu.VMEM((_CHUNKS_PER_W, _CHUNK), jnp.int32),
        pltpu.VMEM((_CHUNK, _DH), jnp.float32),
        pltpu.VMEM((_CHUNK, _DH), jnp.float32),
        pltpu.VMEM((_ZROWS_PER_SUB, _DH), jnp.float32),
        pltpu.VMEM_SHARED((_ACC_ROWS, _DH), jnp.float32),
        pltpu.SemaphoreType.DMA,
        pltpu.SemaphoreType.DMA,
        pltpu.SemaphoreType.DMA,
        pltpu.SemaphoreType.DMA,
    ],
)(_sc_prop_body)


def _tc_mm1_body(x_ref, w_ref, o_ref):
    o_ref[...] = jnp.dot(x_ref[...], w_ref[...],
                         preferred_element_type=jnp.float32)


_tc_mm1 = pl.pallas_call(
    _tc_mm1_body,
    out_shape=jax.ShapeDtypeStruct((_N, _DH), jnp.float32),
)


def _tc_scale1_body(h_ref, degp_ref, ht_ref, dis_ref):
    deg = degp_ref[0, :_N, 0:1] + degp_ref[1, :_N, 0:1] + 1.0
    dis = lax.rsqrt(deg)
    dis_ref[...] = dis
    ht_ref[...] = h_ref[...] * dis


_tc_scale1 = pl.pallas_call(
    _tc_scale1_body,
    out_shape=(
        jax.ShapeDtypeStruct((_N, _DH), jnp.float32),
        jax.ShapeDtypeStruct((_N, 1), jnp.float32),
    ),
)


def _tc_mid_body(accp_ref, ht1_ref, dis_ref, b1_ref, ht2_ref):
    dis = dis_ref[...]
    z = dis * (accp_ref[0, :_N] + accp_ref[1, :_N] + ht1_ref[...]) + b1_ref[...]
    ht2_ref[...] = dis * jnp.maximum(z, 0.0)


_tc_mid = pl.pallas_call(
    _tc_mid_body,
    out_shape=jax.ShapeDtypeStruct((_N, _DH), jnp.float32),
)


def _tc_fin_body(accp_ref, ht2_ref, dis_ref, w2_ref, b2_ref, o_ref):
    t = dis_ref[...] * (accp_ref[0, :_N] + accp_ref[1, :_N] + ht2_ref[...])
    logits = jnp.dot(t, w2_ref[...], preferred_element_type=jnp.float32)
    logits = logits + b2_ref[...]
    m = jnp.max(logits, axis=1, keepdims=True)
    e = jnp.exp(logits - m)
    s = jnp.sum(e, axis=1, keepdims=True)
    o_ref[...] = logits - m - jnp.log(s)


_tc_fin = pl.pallas_call(
    _tc_fin_body,
    out_shape=jax.ShapeDtypeStruct((_N, _DOUT), jnp.float32),
)


def kernel(x, edge_index, W1, b1, W2, b2):
    ei = edge_index.astype(jnp.int32)
    pad = _EPAD - _E
    src = jnp.concatenate([ei[0], jnp.zeros((pad,), jnp.int32)])
    dst = jnp.concatenate([ei[1], jnp.full((pad,), _N, jnp.int32)])
    src3 = src.reshape(_NW, _CHUNKS_PER_W, _CHUNK)
    dst3 = dst.reshape(_NW, _CHUNKS_PER_W, _CHUNK)

    deg_parts = _sc_deg(dst3)
    h1 = _tc_mm1(x, W1)
    ht1, dis = _tc_scale1(h1, deg_parts)
    acc1 = _sc_prop(src3, dst3, ht1)
    ht2 = _tc_mid(acc1, ht1, dis, b1.reshape(1, _DH))
    acc2 = _sc_prop(src3, dst3, ht2)
    return _tc_fin(acc2, ht2, dis, W2, b2.reshape(1, _DOUT))


# speedup vs baseline: 54.5201x; 1.4844x over previous
import functools

import jax
import jax.numpy as jnp
from jax import lax
from jax.experimental import pallas as pl
from jax.experimental.pallas import tpu as pltpu
from jax.experimental.pallas import tpu_sc as plsc

_N = 10000
_E = 320000
_DIN = 128
_DH = 16
_DOUT = 7
_NC = 2
_NS = 16
_NW = _NC * _NS
_CHUNK = 2500
_CHUNKS_PER_W = 4
_ACC_ROWS = 10112
_ZROWS_PER_SUB = _ACC_ROWS // _NS
_OROWS_PER_SUB = _ACC_ROWS // _NS

_MESH = plsc.VectorSubcoreMesh(core_axis_name="c", subcore_axis_name="s")
_SC_PARAMS = pltpu.CompilerParams(use_tc_tiling_on_sc=False)


def _zero_acc_slice(zbuf, acc_sh, sid):
    def zbody(i, carry):
        zbuf[i, :] = jnp.zeros((_DH,), jnp.float32)
        return carry
    lax.fori_loop(0, _ZROWS_PER_SUB, zbody, None)
    pltpu.sync_copy(zbuf, acc_sh.at[pl.ds(sid * _ZROWS_PER_SUB, _ZROWS_PER_SUB)])


def _write_out_slice(acc_sh, out_hbm, cid, sid):
    pltpu.sync_copy(
        acc_sh.at[pl.ds(sid * _OROWS_PER_SUB, _OROWS_PER_SUB)],
        out_hbm.at[cid, pl.ds(sid * _OROWS_PER_SUB, _OROWS_PER_SUB)],
    )


def _sc_deg_body(dst_hbm, out_hbm, idx_d, ones_v, zbuf, acc_sh, dsem):
    cid = lax.axis_index("c")
    sid = lax.axis_index("s")
    wid = cid * _NS + sid
    _zero_acc_slice(zbuf, acc_sh, sid)

    def obody(i, carry):
        ones_v[i, :] = jnp.ones((_DH,), jnp.float32)
        return carry
    lax.fori_loop(0, _CHUNK, obody, None)
    pltpu.sync_copy(dst_hbm.at[wid], idx_d)
    plsc.subcore_barrier()

    for j in range(_CHUNKS_PER_W):
        pltpu.async_copy(ones_v, acc_sh.at[idx_d.at[j]], dsem, add=True)
    for j in range(_CHUNKS_PER_W):
        pltpu.make_async_copy(ones_v, acc_sh.at[idx_d.at[j]], dsem).wait()
    plsc.subcore_barrier()
    _write_out_slice(acc_sh, out_hbm, cid, sid)


_sc_deg = functools.partial(
    pl.kernel,
    out_type=jax.ShapeDtypeStruct((_NC, _ACC_ROWS, _DH), jnp.float32),
    mesh=_MESH,
    compiler_params=_SC_PARAMS,
    scratch_types=[
        pltpu.VMEM((_CHUNKS_PER_W, _CHUNK), jnp.int32),
        pltpu.VMEM((_CHUNK, _DH), jnp.float32),
        pltpu.VMEM((_ZROWS_PER_SUB, _DH), jnp.float32),
        pltpu.VMEM_SHARED((_ACC_ROWS, _DH), jnp.float32),
        pltpu.SemaphoreType.DMA,
    ],
)(_sc_deg_body)


def _sc_prop_body(src_hbm, dst_hbm, h_hbm, out_hbm, idx_s, idx_d, rows_a,
                  rows_b, zbuf, acc_sh, gsem_a, gsem_b, ssem_a, ssem_b):
    cid = lax.axis_index("c")
    sid = lax.axis_index("s")
    wid = cid * _NS + sid
    _zero_acc_slice(zbuf, acc_sh, sid)
    pltpu.sync_copy(src_hbm.at[wid], idx_s)
    pltpu.sync_copy(dst_hbm.at[wid], idx_d)
    plsc.subcore_barrier()

    bufs = (rows_a, rows_b)
    gsems = (gsem_a, gsem_b)
    ssems = (ssem_a, ssem_b)
    nch = _CHUNKS_PER_W
    pltpu.async_copy(h_hbm.at[idx_s.at[0]], bufs[0], gsems[0])
    for j in range(nch):
        b = j % 2
        pltpu.make_async_copy(h_hbm.at[idx_s.at[j]], bufs[b], gsems[b]).wait()
        pltpu.async_copy(bufs[b], acc_sh.at[idx_d.at[j]], ssems[b], add=True)
        if j + 1 < nch:
            nb = (j + 1) % 2
            if j - 1 >= 0:
                pltpu.make_async_copy(
                    bufs[nb], acc_sh.at[idx_d.at[j - 1]], ssems[nb]).wait()
            pltpu.async_copy(h_hbm.at[idx_s.at[j + 1]], bufs[nb], gsems[nb])
    for j in (nch - 2, nch - 1):
        pltpu.make_async_copy(
            bufs[j % 2], acc_sh.at[idx_d.at[j]], ssems[j % 2]).wait()
    plsc.subcore_barrier()
    _write_out_slice(acc_sh, out_hbm, cid, sid)


_sc_prop = functools.partial(
    pl.kernel,
    out_type=jax.ShapeDtypeStruct((_NC, _ACC_ROWS, _DH), jnp.float32),
    mesh=_MESH,
    compiler_params=_SC_PARAMS,
    scratch_types=[
        pltpu.VMEM((_CHUNKS_PER_W, _CHUNK), jnp.int32),
        pltpu.VMEM((_CHUNKS_PER_W, _CHUNK), jnp.int32),
        pltpu.VMEM((_CHUNK, _DH), jnp.float32),
        pltpu.VMEM((_CHUNK, _DH), jnp.float32),
        pltpu.VMEM((_ZROWS_PER_SUB, _DH), jnp.float32),
        pltpu.VMEM_SHARED((_ACC_ROWS, _DH), jnp.float32),
        pltpu.SemaphoreType.DMA,
        pltpu.SemaphoreType.DMA,
        pltpu.SemaphoreType.DMA,
        pltpu.SemaphoreType.DMA,
    ],
)(_sc_prop_body)


def _tc_mm1_body(x_ref, w_ref, o_ref):
    o_ref[...] = jnp.dot(x_ref[...], w_ref[...],
                         preferred_element_type=jnp.float32)


_tc_mm1 = pl.pallas_call(
    _tc_mm1_body,
    out_shape=jax.ShapeDtypeStruct((_N, _DH), jnp.float32),
)


def _tc_scale1_body(h_ref, degp_ref, ht_ref, dis_ref):
    deg = degp_ref[0, :_N, 0:1] + degp_ref[1, :_N, 0:1] + 1.0
    dis = lax.rsqrt(deg)
    dis_ref[...] = dis
    ht_ref[...] = h_ref[...] * dis


_tc_scale1 = pl.pallas_call(
    _tc_scale1_body,
    out_shape=(
        jax.ShapeDtypeStruct((_N, _DH), jnp.float32),
        jax.ShapeDtypeStruct((_N, 1), jnp.float32),
    ),
)


def _tc_mid_body(accp_ref, ht1_ref, dis_ref, b1_ref, ht2_ref):
    dis = dis_ref[...]
    z = dis * (accp_ref[0, :_N] + accp_ref[1, :_N] + ht1_ref[...]) + b1_ref[...]
    ht2_ref[...] = dis * jnp.maximum(z, 0.0)


_tc_mid = pl.pallas_call(
    _tc_mid_body,
    out_shape=jax.ShapeDtypeStruct((_N, _DH), jnp.float32),
)


def _tc_fin_body(accp_ref, ht2_ref, dis_ref, w2_ref, b2_ref, o_ref):
    t = dis_ref[...] * (accp_ref[0, :_N] + accp_ref[1, :_N] + ht2_ref[...])
    logits = jnp.dot(t, w2_ref[...], preferred_element_type=jnp.float32)
    logits = logits + b2_ref[...]
    m = jnp.max(logits, axis=1, keepdims=True)
    e = jnp.exp(logits - m)
    s = jnp.sum(e, axis=1, keepdims=True)
    o_ref[...] = logits - m - jnp.log(s)


_tc_fin = pl.pallas_call(
    _tc_fin_body,
    out_shape=jax.ShapeDtypeStruct((_N, _DOUT), jnp.float32),
)


def kernel(x, edge_index, W1, b1, W2, b2):
    ei = edge_index.astype(jnp.int32)
    src3 = ei[0].reshape(_NW, _CHUNKS_PER_W, _CHUNK)
    dst3 = ei[1].reshape(_NW, _CHUNKS_PER_W, _CHUNK)

    deg_parts = _sc_deg(dst3)
    h1 = _tc_mm1(x, W1)
    ht1, dis = _tc_scale1(h1, deg_parts)
    acc1 = _sc_prop(src3, dst3, ht1)
    ht2 = _tc_mid(acc1, ht1, dis, b1.reshape(1, _DH))
    acc2 = _sc_prop(src3, dst3, ht2)
    return _tc_fin(acc2, ht2, dis, W2, b2.reshape(1, _DOUT))


# speedup vs baseline: 55.3981x; 1.0161x over previous
import functools

import jax
import jax.numpy as jnp
from jax import lax
from jax.experimental import pallas as pl
from jax.experimental.pallas import tpu as pltpu
from jax.experimental.pallas import tpu_sc as plsc

_N = 10000
_E = 320000
_DIN = 128
_DH = 16
_DOUT = 7
_NC = 2
_NS = 16
_NW = _NC * _NS
_CHUNK = 2000
_NCH = 5
_EPW = _NCH * _CHUNK
_ACC_ROWS = 10112
_RPS = _ACC_ROWS // _NS

_MESH = plsc.VectorSubcoreMesh(core_axis_name="c", subcore_axis_name="s")
_SC_PARAMS = pltpu.CompilerParams(use_tc_tiling_on_sc=False,
                                  needs_layout_passes=False)


def _zero_acc_slice(zbuf, acc_sh, sid):
    def zbody(i, carry):
        zbuf[i, :] = jnp.zeros((_DH,), jnp.float32)
        return carry
    lax.fori_loop(0, _RPS, zbody, None)
    pltpu.sync_copy(zbuf, acc_sh.at[pl.ds(sid * _RPS, _RPS)])


def _load_slab(ei_hbm, row, slab, idx):
    for j in range(_NCH):
        pltpu.sync_copy(ei_hbm.at[row, pl.ds(slab * _EPW + j * _CHUNK, _CHUNK)],
                        idx.at[j])


def _propagate(h_sh, acc_sh, idx_s, idx_d, rows_a, rows_b,
               gsem_a, gsem_b, ssem_a, ssem_b):
    bufs = (rows_a, rows_b)
    gsems = (gsem_a, gsem_b)
    ssems = (ssem_a, ssem_b)
    pltpu.async_copy(h_sh.at[idx_s.at[0]], bufs[0], gsems[0])
    for j in range(_NCH):
        b = j % 2
        pltpu.make_async_copy(h_sh.at[idx_s.at[j]], bufs[b], gsems[b]).wait()
        pltpu.async_copy(bufs[b], acc_sh.at[idx_d.at[j]], ssems[b], add=True)
        if j + 1 < _NCH:
            nb = (j + 1) % 2
            if j - 1 >= 0:
                pltpu.make_async_copy(
                    bufs[nb], acc_sh.at[idx_d.at[j - 1]], ssems[nb]).wait()
            pltpu.async_copy(h_sh.at[idx_s.at[j + 1]], bufs[nb], gsems[nb])
    for j in (_NCH - 2, _NCH - 1):
        pltpu.make_async_copy(
            bufs[j % 2], acc_sh.at[idx_d.at[j]], ssems[j % 2]).wait()


def _sc_l1_body(ei_hbm, h1_hbm, accp, ht1_hbm, disr_hbm,
                idx_s, idx_d, rows_a, rows_b, zbuf, acc_sh, h_sh,
                gsem_a, gsem_b, ssem_a, ssem_b, dsem):
    cid = lax.axis_index("c")
    sid = lax.axis_index("s")
    wid = cid * _NS + sid
    rsl = pl.ds(sid * _RPS, _RPS)
    _zero_acc_slice(zbuf, acc_sh, sid)

    def obody(i, carry):
        rows_a[i, :] = jnp.ones((_DH,), jnp.float32)
        return carry
    lax.fori_loop(0, _CHUNK, obody, None)
    _load_slab(ei_hbm, 1, sid, idx_s)
    _load_slab(ei_hbm, 1, sid + _NS, idx_d)
    plsc.subcore_barrier()
    for idx in (idx_s, idx_d):
        for j in range(_NCH):
            pltpu.async_copy(rows_a, acc_sh.at[idx.at[j]], dsem, add=True)
    for idx in (idx_s, idx_d):
        for j in range(_NCH):
            pltpu.make_async_copy(rows_a, acc_sh.at[idx.at[j]], dsem).wait()
    plsc.subcore_barrier()

    pltpu.sync_copy(acc_sh.at[rsl], zbuf)

    def dbody(i, carry):
        d = zbuf[i, :] + 1.0
        y = plsc.bitcast(1597463007 - (plsc.bitcast(d, jnp.int32) >> 1),
                         jnp.float32)
        y = y * (1.5 - 0.5 * d * y * y)
        y = y * (1.5 - 0.5 * d * y * y)
        y = y * (1.5 - 0.5 * d * y * y)
        zbuf[i, :] = y
        return carry
    lax.fori_loop(0, _RPS, dbody, None)

    @pl.when(cid == 0)
    def _():
        pltpu.sync_copy(zbuf, disr_hbm.at[rsl])

    pltpu.sync_copy(h1_hbm.at[rsl], rows_b.at[pl.ds(0, _RPS)])

    def sbody(i, carry):
        rows_b[i, :] = rows_b[i, :] * zbuf[i, :]
        return carry
    lax.fori_loop(0, _RPS, sbody, None)
    pltpu.sync_copy(rows_b.at[pl.ds(0, _RPS)], h_sh.at[rsl])

    @pl.when(cid == 0)
    def _():
        pltpu.sync_copy(rows_b.at[pl.ds(0, _RPS)], ht1_hbm.at[rsl])

    _zero_acc_slice(zbuf, acc_sh, sid)
    _load_slab(ei_hbm, 0, wid, idx_s)
    _load_slab(ei_hbm, 1, wid, idx_d)
    plsc.subcore_barrier()

    _propagate(h_sh, acc_sh, idx_s, idx_d, rows_a, rows_b,
               gsem_a, gsem_b, ssem_a, ssem_b)
    plsc.subcore_barrier()
    pltpu.sync_copy(acc_sh.at[rsl], accp.at[cid, rsl])


_sc_l1 = functools.partial(
    pl.kernel,
    out_type=(
        jax.ShapeDtypeStruct((_NC, _ACC_ROWS, _DH), jnp.float32),
        jax.ShapeDtypeStruct((_ACC_ROWS, _DH), jnp.float32),
        jax.ShapeDtypeStruct((_ACC_ROWS, _DH), jnp.float32),
    ),
    mesh=_MESH,
    compiler_params=_SC_PARAMS,
    scratch_types=[
        pltpu.VMEM((_NCH, _CHUNK), jnp.int32),
        pltpu.VMEM((_NCH, _CHUNK), jnp.int32),
        pltpu.VMEM((_CHUNK, _DH), jnp.float32),
        pltpu.VMEM((_CHUNK, _DH), jnp.float32),
        pltpu.VMEM((_RPS, _DH), jnp.float32),
        pltpu.VMEM_SHARED((_ACC_ROWS, _DH), jnp.float32),
        pltpu.VMEM_SHARED((_ACC_ROWS, _DH), jnp.float32),
        pltpu.SemaphoreType.DMA,
        pltpu.SemaphoreType.DMA,
        pltpu.SemaphoreType.DMA,
        pltpu.SemaphoreType.DMA,
        pltpu.SemaphoreType.DMA,
    ],
)(_sc_l1_body)


def _sc_l2_body(ei_hbm, accp1, ht1_hbm, disr_hbm, b1_hbm, accp, ht2_hbm,
                idx_s, idx_d, rows_a, rows_b, zbuf, bbuf, acc_sh, h_sh,
                gsem_a, gsem_b, ssem_a, ssem_b):
    cid = lax.axis_index("c")
    sid = lax.axis_index("s")
    wid = cid * _NS + sid
    rsl = pl.ds(sid * _RPS, _RPS)
    _zero_acc_slice(zbuf, acc_sh, sid)

    pltpu.sync_copy(b1_hbm, bbuf)
    pltpu.sync_copy(disr_hbm.at[rsl], zbuf)
    pltpu.sync_copy(accp1.at[0, rsl], rows_a.at[pl.ds(0, _RPS)])
    pltpu.sync_copy(accp1.at[1, rsl], rows_a.at[pl.ds(_RPS, _RPS)])
    pltpu.sync_copy(ht1_hbm.at[rsl], rows_b.at[pl.ds(0, _RPS)])
    b1v = bbuf[...]

    def mbody(i, carry):
        dis = zbuf[i, :]
        z = dis * (rows_a[i, :] + rows_a[_RPS + i, :] + rows_b[i, :]) + b1v
        rows_b[i, :] = dis * jnp.maximum(z, 0.0)
        return carry
    lax.fori_loop(0, _RPS, mbody, None)
    pltpu.sync_copy(rows_b.at[pl.ds(0, _RPS)], h_sh.at[rsl])

    @pl.when(cid == 0)
    def _():
        pltpu.sync_copy(rows_b.at[pl.ds(0, _RPS)], ht2_hbm.at[rsl])

    _load_slab(ei_hbm, 0, wid, idx_s)
    _load_slab(ei_hbm, 1, wid, idx_d)
    plsc.subcore_barrier()

    _propagate(h_sh, acc_sh, idx_s, idx_d, rows_a, rows_b,
               gsem_a, gsem_b, ssem_a, ssem_b)
    plsc.subcore_barrier()
    pltpu.sync_copy(acc_sh.at[rsl], accp.at[cid, rsl])


_sc_l2 = functools.partial(
    pl.kernel,
    out_type=(
        jax.ShapeDtypeStruct((_NC, _ACC_ROWS, _DH), jnp.float32),
        jax.ShapeDtypeStruct((_ACC_ROWS, _DH), jnp.float32),
    ),
    mesh=_MESH,
    compiler_params=_SC_PARAMS,
    scratch_types=[
        pltpu.VMEM((_NCH, _CHUNK), jnp.int32),
        pltpu.VMEM((_NCH, _CHUNK), jnp.int32),
        pltpu.VMEM((_CHUNK, _DH), jnp.float32),
        pltpu.VMEM((_CHUNK, _DH), jnp.float32),
        pltpu.VMEM((_RPS, _DH), jnp.float32),
        pltpu.VMEM((_DH,), jnp.float32),
        pltpu.VMEM_SHARED((_ACC_ROWS, _DH), jnp.float32),
        pltpu.VMEM_SHARED((_ACC_ROWS, _DH), jnp.float32),
        pltpu.SemaphoreType.DMA,
        pltpu.SemaphoreType.DMA,
        pltpu.SemaphoreType.DMA,
        pltpu.SemaphoreType.DMA,
    ],
)(_sc_l2_body)


def _tc_mm1_body(x_ref, w_ref, o_ref):
    o_ref[:_N, :] = jnp.dot(x_ref[...], w_ref[...],
                            preferred_element_type=jnp.float32)
    o_ref[_N:, :] = jnp.zeros((_ACC_ROWS - _N, _DH), jnp.float32)


_tc_mm1 = pl.pallas_call(
    _tc_mm1_body,
    out_shape=jax.ShapeDtypeStruct((_ACC_ROWS, _DH), jnp.float32),
)


def _tc_fin_body(accp_ref, ht2_ref, disr_ref, w2_ref, b2_ref, o_ref):
    t = disr_ref[:_N] * (accp_ref[0, :_N] + accp_ref[1, :_N] + ht2_ref[:_N])
    logits = jnp.dot(t, w2_ref[...], preferred_element_type=jnp.float32)
    logits = logits + b2_ref[...]
    m = jnp.max(logits, axis=1, keepdims=True)
    e = jnp.exp(logits - m)
    s = jnp.sum(e, axis=1, keepdims=True)
    o_ref[...] = logits - m - jnp.log(s)


_tc_fin = pl.pallas_call(
    _tc_fin_body,
    out_shape=jax.ShapeDtypeStruct((_N, _DOUT), jnp.float32),
)


def kernel(x, edge_index, W1, b1, W2, b2):
    ei = edge_index.astype(jnp.int32)
    h1 = _tc_mm1(x, W1)
    acc1, ht1, disr = _sc_l1(ei, h1)
    acc2, ht2 = _sc_l2(ei, acc1, ht1, disr, b1)
    return _tc_fin(acc2, ht2, disr, W2, b2.reshape(1, _DOUT))


# speedup vs baseline: 84.0246x; 1.5167x over previous
import functools

import jax
import jax.numpy as jnp
from jax import lax
from jax.experimental import pallas as pl
from jax.experimental.pallas import tpu as pltpu
from jax.experimental.pallas import tpu_sc as plsc

_N = 10000
_E = 320000
_DIN = 128
_DH = 16
_DOUT = 7
_NC = 2
_NS = 16
_NW = _NC * _NS
_CHUNK = 1000
_NCH = 10
_EPW = _NCH * _CHUNK
_ACC_ROWS = 10112
_WROWS = _ACC_ROWS // 8
_WR = _WROWS // _NS
_RPS = _ACC_ROWS // _NS
_RB = 2 * _RPS

_MESH = plsc.VectorSubcoreMesh(core_axis_name="c", subcore_axis_name="s")
_SC_PARAMS = pltpu.CompilerParams(use_tc_tiling_on_sc=False,
                                  needs_layout_passes=False)


def _row_loop(n, body, unroll=8):
    def ubody(k, carry):
        for u in range(unroll):
            body(k * unroll + u)
        return carry
    lax.fori_loop(0, n // unroll, ubody, None)


def _fill_const(buf, n, val):
    def body(i):
        buf[i, :] = jnp.full((_DH,), val, jnp.float32)
    _row_loop(n, body)


def _load_slab_start(ei_hbm, row, slab, idx, sem):
    for j in range(_NCH):
        pltpu.async_copy(
            ei_hbm.at[row, pl.ds(slab * _EPW + j * _CHUNK, _CHUNK)],
            idx.at[j], sem)


def _load_slab_wait(ei_hbm, row, slab, idx, sem):
    for j in range(_NCH):
        pltpu.make_async_copy(
            ei_hbm.at[row, pl.ds(slab * _EPW + j * _CHUNK, _CHUNK)],
            idx.at[j], sem).wait()


def _writeback(cid, rsl, acc_sh, h_sh, rows_a, rows_b, accp):
    @pl.when(cid == 0)
    def _():
        pltpu.sync_copy(acc_sh.at[rsl], rows_a.at[pl.ds(0, _RPS)])
        pltpu.sync_copy(h_sh.at[rsl], rows_b.at[pl.ds(0, _RPS)])

        def abody(i):
            rows_a[i, :] = rows_a[i, :] + rows_b[i, :]
        _row_loop(_RPS, abody)
        pltpu.sync_copy(rows_a.at[pl.ds(0, _RPS)], accp.at[0, rsl])

    @pl.when(cid == 1)
    def _():
        pltpu.sync_copy(acc_sh.at[rsl], accp.at[1, rsl])


def _propagate(h_sh, acc_sh, idx_s, idx_d, rows_a, rows_b,
               gsem_a, gsem_b, ssem_a, ssem_b):
    bufs = (rows_a.at[pl.ds(0, _CHUNK)], rows_b.at[pl.ds(0, _CHUNK)])
    gsems = (gsem_a, gsem_b)
    ssems = (ssem_a, ssem_b)
    pltpu.async_copy(h_sh.at[idx_s.at[0]], bufs[0], gsems[0])
    for j in range(_NCH):
        b = j % 2
        pltpu.make_async_copy(h_sh.at[idx_s.at[j]], bufs[b], gsems[b]).wait()
        pltpu.async_copy(bufs[b], acc_sh.at[idx_d.at[j]], ssems[b], add=True)
        if j + 1 < _NCH:
            nb = (j + 1) % 2
            if j - 1 >= 0:
                pltpu.make_async_copy(
                    bufs[nb], acc_sh.at[idx_d.at[j - 1]], ssems[nb]).wait()
            pltpu.async_copy(h_sh.at[idx_s.at[j + 1]], bufs[nb], gsems[nb])
    for j in (_NCH - 2, _NCH - 1):
        pltpu.make_async_copy(
            bufs[j % 2], acc_sh.at[idx_d.at[j]], ssems[j % 2]).wait()


def _sc_l1_body(ei_hbm, h1_hbm, accp, disr_hbm,
                idx_s, idx_d, rows_a, rows_b, dbuf, acc_sh, h_sh,
                gsem_a, gsem_b, ssem_a, ssem_b, dsem):
    cid = lax.axis_index("c")
    sid = lax.axis_index("s")
    wid = cid * _NS + sid
    rsl = pl.ds(sid * _RPS, _RPS)

    _load_slab_start(ei_hbm, 1, sid, idx_s, dsem)
    _load_slab_start(ei_hbm, 1, sid + _NS, idx_d, dsem)
    pltpu.async_copy(h1_hbm.at[rsl], rows_b.at[pl.ds(0, _RPS)], gsem_a)

    _fill_const(rows_b, _RPS, 0.0)
    pltpu.sync_copy(rows_b.at[pl.ds(0, _RPS)], acc_sh.at[rsl])
    _fill_const(rows_a, _CHUNK, 1.0)
    _load_slab_wait(ei_hbm, 1, sid, idx_s, dsem)
    _load_slab_wait(ei_hbm, 1, sid + _NS, idx_d, dsem)
    plsc.subcore_barrier()
    ones = rows_a.at[pl.ds(0, _CHUNK)]
    for idx in (idx_s, idx_d):
        for j in range(_NCH):
            pltpu.async_copy(ones, acc_sh.at[idx.at[j]], dsem, add=True)
    for idx in (idx_s, idx_d):
        for j in range(_NCH):
            pltpu.make_async_copy(ones, acc_sh.at[idx.at[j]], dsem).wait()
    plsc.subcore_barrier()

    pltpu.sync_copy(acc_sh.at[rsl], dbuf)
    pltpu.sync_copy(rows_b.at[pl.ds(0, _RPS)], acc_sh.at[rsl])
    _load_slab_start(ei_hbm, 0, wid, idx_s, dsem)
    _load_slab_start(ei_hbm, 1, wid, idx_d, dsem)

    def dbody(i):
        d = dbuf[i, :] + 1.0
        y = plsc.bitcast(1597463007 - (plsc.bitcast(d, jnp.int32) >> 1),
                         jnp.float32)
        y = y * (1.5 - 0.5 * d * y * y)
        y = y * (1.5 - 0.5 * d * y * y)
        y = y * (1.5 - 0.5 * d * y * y)
        dbuf[i, :] = y
    _row_loop(_RPS, dbody)

    @pl.when(cid == 0)
    def _():
        pltpu.sync_copy(dbuf, disr_hbm.at[rsl])

    pltpu.make_async_copy(h1_hbm.at[rsl], rows_b.at[pl.ds(0, _RPS)],
                          gsem_a).wait()

    def sbody(i):
        rows_b[i, :] = rows_b[i, :] * dbuf[i, :]
    _row_loop(_RPS, sbody)
    pltpu.sync_copy(rows_b.at[pl.ds(0, _RPS)], h_sh.at[rsl])

    _load_slab_wait(ei_hbm, 0, wid, idx_s, dsem)
    _load_slab_wait(ei_hbm, 1, wid, idx_d, dsem)
    plsc.subcore_barrier()

    _propagate(h_sh, acc_sh, idx_s, idx_d, rows_a, rows_b,
               gsem_a, gsem_b, ssem_a, ssem_b)
    plsc.subcore_barrier()
    _writeback(cid, rsl, acc_sh, h_sh, rows_a, rows_b, accp)


_sc_l1 = functools.partial(
    pl.kernel,
    out_type=(
        jax.ShapeDtypeStruct((_NC, _ACC_ROWS, _DH), jnp.float32),
        jax.ShapeDtypeStruct((_ACC_ROWS, _DH), jnp.float32),
    ),
    mesh=_MESH,
    compiler_params=_SC_PARAMS,
    scratch_types=[
        pltpu.VMEM((_NCH, _CHUNK), jnp.int32),
        pltpu.VMEM((_NCH, _CHUNK), jnp.int32),
        pltpu.VMEM((_RB, _DH), jnp.float32),
        pltpu.VMEM((_RB, _DH), jnp.float32),
        pltpu.VMEM((_RPS, _DH), jnp.float32),
        pltpu.VMEM_SHARED((_ACC_ROWS, _DH), jnp.float32),
        pltpu.VMEM_SHARED((_ACC_ROWS, _DH), jnp.float32),
        pltpu.SemaphoreType.DMA,
        pltpu.SemaphoreType.DMA,
        pltpu.SemaphoreType.DMA,
        pltpu.SemaphoreType.DMA,
        pltpu.SemaphoreType.DMA,
    ],
)(_sc_l1_body)


def _sc_l2_body(ei_hbm, accp1, disr_hbm, b1_hbm, accp,
                idx_s, idx_d, rows_a, rows_b, dbuf, wbuf, bbuf,
                acc_sh, h_sh, gsem_a, gsem_b, ssem_a, ssem_b, dsem):
    cid = lax.axis_index("c")
    sid = lax.axis_index("s")
    wid = cid * _NS + sid
    rsl = pl.ds(sid * _RPS, _RPS)

    pltpu.async_copy(b1_hbm, bbuf, gsem_a)
    pltpu.async_copy(disr_hbm.at[rsl], dbuf, gsem_a)
    pltpu.async_copy(accp1.at[0, rsl], rows_a.at[pl.ds(0, _RPS)], gsem_a)
    pltpu.async_copy(accp1.at[1, rsl], rows_a.at[pl.ds(_RPS, _RPS)], gsem_a)
    _load_slab_start(ei_hbm, 0, wid, idx_s, dsem)
    _load_slab_start(ei_hbm, 1, wid, idx_d, dsem)

    _fill_const(rows_b, _RPS, 0.0)
    pltpu.sync_copy(rows_b.at[pl.ds(0, _RPS)], acc_sh.at[rsl])

    pltpu.make_async_copy(b1_hbm, bbuf, gsem_a).wait()
    pltpu.make_async_copy(disr_hbm.at[rsl], dbuf, gsem_a).wait()
    pltpu.make_async_copy(accp1.at[0, rsl], rows_a.at[pl.ds(0, _RPS)],
                          gsem_a).wait()
    pltpu.make_async_copy(accp1.at[1, rsl], rows_a.at[pl.ds(_RPS, _RPS)],
                          gsem_a).wait()
    b1v = bbuf[...]

    def mbody(i):
        dis = dbuf[i, :]
        z = dis * (rows_a[i, :] + rows_a[_RPS + i, :]) + b1v
        rows_b[i, :] = dis * jnp.maximum(z, 0.0)
    _row_loop(_RPS, mbody)
    pltpu.sync_copy(rows_b.at[pl.ds(0, _RPS)], h_sh.at[rsl])

    _load_slab_wait(ei_hbm, 0, wid, idx_s, dsem)
    _load_slab_wait(ei_hbm, 1, wid, idx_d, dsem)
    plsc.subcore_barrier()

    _propagate(h_sh, acc_sh, idx_s, idx_d, rows_a, rows_b,
               gsem_a, gsem_b, ssem_a, ssem_b)
    plsc.subcore_barrier()
    pltpu.sync_copy(acc_sh.at[rsl], rows_a.at[pl.ds(0, _RPS)])

    @pl.when(cid == 0)
    def _():
        pltpu.sync_copy(h_sh.at[rsl], rows_b.at[pl.ds(0, _RPS)])

        def abody(i):
            rows_a[i, :] = rows_a[i, :] + rows_b[i, :]
        _row_loop(_RPS, abody)

    def tbody(i):
        rows_a[i, :] = rows_a[i, :] * dbuf[i, :]
    _row_loop(_RPS, tbody)

    def pbody(k, carry):
        for u in range(8):
            wbuf[k, pl.ds(16 * u, _DH)] = rows_a[8 * k + u, :]
        return carry
    lax.fori_loop(0, _WR, pbody, None)
    pltpu.sync_copy(wbuf, accp.at[cid, pl.ds(sid * _WR, _WR)])


_sc_l2 = functools.partial(
    pl.kernel,
    out_type=jax.ShapeDtypeStruct((_NC, _WROWS, 128), jnp.float32),
    mesh=_MESH,
    compiler_params=_SC_PARAMS,
    scratch_types=[
        pltpu.VMEM((_NCH, _CHUNK), jnp.int32),
        pltpu.VMEM((_NCH, _CHUNK), jnp.int32),
        pltpu.VMEM((_RB, _DH), jnp.float32),
        pltpu.VMEM((_RB, _DH), jnp.float32),
        pltpu.VMEM((_RPS, _DH), jnp.float32),
        pltpu.VMEM((_WR, 128), jnp.float32),
        pltpu.VMEM((_DH,), jnp.float32),
        pltpu.VMEM_SHARED((_ACC_ROWS, _DH), jnp.float32),
        pltpu.VMEM_SHARED((_ACC_ROWS, _DH), jnp.float32),
        pltpu.SemaphoreType.DMA,
        pltpu.SemaphoreType.DMA,
        pltpu.SemaphoreType.DMA,
        pltpu.SemaphoreType.DMA,
        pltpu.SemaphoreType.DMA,
    ],
)(_sc_l2_body)


def _tc_mm1_body(x_ref, w_ref, o_ref):
    o_ref[:_N, :] = jnp.dot(x_ref[...], w_ref[...],
                            preferred_element_type=jnp.float32)
    o_ref[_N:, :] = jnp.zeros((_ACC_ROWS - _N, _DH), jnp.float32)


_tc_mm1 = pl.pallas_call(
    _tc_mm1_body,
    out_shape=jax.ShapeDtypeStruct((_ACC_ROWS, _DH), jnp.float32),
)


def _tc_fin_body(accp_ref, w2bd_ref, b2t_ref, o_ref):
    tw = accp_ref[0] + accp_ref[1]
    o_ref[...] = jnp.dot(tw, w2bd_ref[...],
                         preferred_element_type=jnp.float32) + b2t_ref[...]


_tc_fin = pl.pallas_call(
    _tc_fin_body,
    out_shape=jax.ShapeDtypeStruct((_WROWS, 8 * _DOUT), jnp.float32),
)


def _tc_sm_body(l_ref, o_ref):
    logits = l_ref[:_N]
    m = jnp.max(logits, axis=1, keepdims=True)
    e = jnp.exp(logits - m)
    s = jnp.sum(e, axis=1, keepdims=True)
    o_ref[...] = logits - m - jnp.log(s)


_tc_sm = pl.pallas_call(
    _tc_sm_body,
    out_shape=jax.ShapeDtypeStruct((_N, _DOUT), jnp.float32),
)


def kernel(x, edge_index, W1, b1, W2, b2):
    ei = edge_index.astype(jnp.int32)
    h1 = _tc_mm1(x, W1)
    acc1, disr = _sc_l1(ei, h1)
    acc2 = _sc_l2(ei, acc1, disr, b1)
    w2bd = jnp.kron(jnp.eye(8, dtype=jnp.float32), W2)
    b2t = jnp.tile(b2, 8).reshape(1, 8 * _DOUT)
    lp = _tc_fin(acc2, w2bd, b2t)
    return _tc_sm(lp.reshape(_ACC_ROWS, _DOUT))
